# Initial kernel scaffold; baseline (speedup 1.0000x reference)
import jax
import jax.numpy as jnp
from jax.experimental import pallas as pl


def kernel(x_user, x_item, edge_index_user_item, edge_index_item_user, W_emb_user, b_emb_user, W_emb_item, b_emb_item, W1_ui, b1_ui, W2_ui, b2_ui, W1_iu, b1_iu, W2_iu, b2_iu):
    raise NotImplementedError("write your pallas kernel here")


# speedup vs baseline: 4.7303x; 4.7303x over previous
import functools

import jax
import jax.numpy as jnp
from jax import lax
from jax.experimental import pallas as pl
from jax.experimental.pallas import tpu as pltpu
from jax.experimental.pallas import tpu_sc as plsc

N = 10000
D = 128
E = 320000

NC = 2
NS = 16
ET = E // NS
K = 80
NCHUNK = ET // K
N_PAD = 10240
ROWS_PER_TILE = N_PAD // NS
ZR = 128


BLK = 2000


def _embed_body(xu_ref, xi_ref, wu_ref, bu_ref, wi_ref, bi_ref,
                hu_ref, hi_ref):
    hu_ref[...] = (
        jnp.dot(xu_ref[...], wu_ref[...], preferred_element_type=jnp.float32)
        + bu_ref[...]
    )
    hi_ref[...] = (
        jnp.dot(xi_ref[...], wi_ref[...], preferred_element_type=jnp.float32)
        + bi_ref[...]
    )


def _embed(x_user, x_item, W_emb_user, b_emb_user, W_emb_item, b_emb_item):
    grid = (N // BLK,)
    row_spec = pl.BlockSpec((BLK, D), lambda i: (i, 0))
    full_spec = pl.BlockSpec((D, D), lambda i: (0, 0))
    bias_spec = pl.BlockSpec((1, D), lambda i: (0, 0))
    return pl.pallas_call(
        _embed_body,
        grid=grid,
        in_specs=[row_spec, row_spec, full_spec, bias_spec, full_spec,
                  bias_spec],
        out_specs=[row_spec, row_spec],
        out_shape=[
            jax.ShapeDtypeStruct((N, D), jnp.float32),
            jax.ShapeDtypeStruct((N, D), jnp.float32),
        ],
    )(x_user, x_item, W_emb_user, b_emb_user.reshape(1, D), W_emb_item,
      b_emb_item.reshape(1, D))


def _sc_agg_body(hu_hbm, hi_hbm, src_ui_hbm, dst_ui_hbm, src_iu_hbm,
                 dst_iu_hbm, agg_item_hbm, agg_user_hbm,
                 idx_src, idx_dst, rows, zbuf, acc):
    core = lax.axis_index("c")
    sub = lax.axis_index("s")

    def _process(h_hbm, src_hbm, dst_hbm, out_hbm):
        def _zero_buf(i, _):
            r = i // (D // 16)
            c = (i % (D // 16)) * 16
            zbuf[r, pl.ds(c, 16)] = jnp.zeros((16,), jnp.float32)
            return ()

        lax.fori_loop(0, ZR * (D // 16), _zero_buf, ())
        base_row = sub * ROWS_PER_TILE

        def _zero_acc(j, _):
            pltpu.sync_copy(zbuf, acc.at[pl.ds(base_row + j * ZR, ZR)])
            return ()

        lax.fori_loop(0, ROWS_PER_TILE // ZR, _zero_acc, ())
        plsc.subcore_barrier()

        base_edge = sub * ET

        def _chunk(g, _):
            off = base_edge + g * K
            pltpu.sync_copy(src_hbm.at[pl.ds(off, K)], idx_src)
            pltpu.sync_copy(dst_hbm.at[pl.ds(off, K)], idx_dst)
            pltpu.sync_copy(h_hbm.at[idx_src], rows)
            pltpu.sync_copy(rows, acc.at[idx_dst], add=True)
            return ()

        lax.fori_loop(0, NCHUNK, _chunk, ())
        plsc.subcore_barrier()

        pltpu.sync_copy(acc.at[pl.ds(base_row, ROWS_PER_TILE)],
                        out_hbm.at[pl.ds(base_row, ROWS_PER_TILE)])

    @pl.when(core == 0)
    def _():
        _process(hu_hbm, src_ui_hbm, dst_ui_hbm, agg_item_hbm)

    @pl.when(core == 1)
    def _():
        _process(hi_hbm, src_iu_hbm, dst_iu_hbm, agg_user_hbm)


def _sc_aggregate(h_user, h_item, src_ui, dst_ui, src_iu, dst_iu):
    mesh = plsc.VectorSubcoreMesh(core_axis_name="c", subcore_axis_name="s",
                                  num_cores=NC, num_subcores=NS)
    agg = pl.kernel(
        _sc_agg_body,
        out_type=[
            jax.ShapeDtypeStruct((N_PAD, D), jnp.float32),
            jax.ShapeDtypeStruct((N_PAD, D), jnp.float32),
        ],
        mesh=mesh,
        scratch_types=[
            pltpu.VMEM((K,), jnp.int32),
            pltpu.VMEM((K,), jnp.int32),
            pltpu.VMEM((K, D), jnp.float32),
            pltpu.VMEM((ZR, D), jnp.float32),
            pltpu.VMEM_SHARED((N_PAD, D), jnp.float32),
        ],
    )
    return agg(h_user, h_item, src_ui, dst_ui, src_iu, dst_iu)


def _mlp_body(hi_ref, ai_ref, hu_ref, au_ref,
              w1ui_ref, b1ui_ref, w2ui_ref, b2ui_ref,
              w1iu_ref, b1iu_ref, w2iu_ref, b2iu_ref,
              oi_ref, ou_ref):
    zi = hi_ref[...] + ai_ref[...]
    ti = jnp.maximum(
        jnp.dot(zi, w1ui_ref[...], preferred_element_type=jnp.float32)
        + b1ui_ref[...], 0.0)
    oi_ref[...] = (
        jnp.dot(ti, w2ui_ref[...], preferred_element_type=jnp.float32)
        + b2ui_ref[...]
    )
    zu = hu_ref[...] + au_ref[...]
    tu = jnp.maximum(
        jnp.dot(zu, w1iu_ref[...], preferred_element_type=jnp.float32)
        + b1iu_ref[...], 0.0)
    ou_ref[...] = (
        jnp.dot(tu, w2iu_ref[...], preferred_element_type=jnp.float32)
        + b2iu_ref[...]
    )


def _mlp(h_item, agg_item, h_user, agg_user,
         W1_ui, b1_ui, W2_ui, b2_ui, W1_iu, b1_iu, W2_iu, b2_iu):
    grid = (N // BLK,)
    row_spec = pl.BlockSpec((BLK, D), lambda i: (i, 0))
    full_spec = pl.BlockSpec((D, D), lambda i: (0, 0))
    bias_spec = pl.BlockSpec((1, D), lambda i: (0, 0))
    return pl.pallas_call(
        _mlp_body,
        grid=grid,
        in_specs=[row_spec, row_spec, row_spec, row_spec,
                  full_spec, bias_spec, full_spec, bias_spec,
                  full_spec, bias_spec, full_spec, bias_spec],
        out_specs=[row_spec, row_spec],
        out_shape=[
            jax.ShapeDtypeStruct((N, D), jnp.float32),
            jax.ShapeDtypeStruct((N, D), jnp.float32),
        ],
    )(h_item, agg_item, h_user, agg_user,
      W1_ui, b1_ui.reshape(1, D), W2_ui, b2_ui.reshape(1, D),
      W1_iu, b1_iu.reshape(1, D), W2_iu, b2_iu.reshape(1, D))


def kernel(x_user, x_item, edge_index_user_item, edge_index_item_user,
           W_emb_user, b_emb_user, W_emb_item, b_emb_item,
           W1_ui, b1_ui, W2_ui, b2_ui, W1_iu, b1_iu, W2_iu, b2_iu):
    src_ui = edge_index_user_item[0]
    dst_ui = edge_index_user_item[1]
    src_iu = edge_index_item_user[0]
    dst_iu = edge_index_item_user[1]

    h_user, h_item = _embed(x_user, x_item, W_emb_user, b_emb_user,
                            W_emb_item, b_emb_item)
    agg_item, agg_user = _sc_aggregate(h_user, h_item, src_ui, dst_ui,
                                       src_iu, dst_iu)
    out_item, out_user = _mlp(h_item, agg_item, h_user, agg_user,
                              W1_ui, b1_ui, W2_ui, b2_ui,
                              W1_iu, b1_iu, W2_iu, b2_iu)
    return (out_user, out_item)


# speedup vs baseline: 9.8392x; 2.0800x over previous
import functools

import jax
import jax.numpy as jnp
from jax import lax
from jax.experimental import pallas as pl
from jax.experimental.pallas import tpu as pltpu
from jax.experimental.pallas import tpu_sc as plsc

N = 10000
D = 128
E = 320000

NC = 2
NS = 16
ET = E // NS
K = 125
NCHUNK = ET // K
N_PAD = 10240
ROWS_PER_TILE = N_PAD // NS
ZR = 64


BLK = 2000


def _embed_body(xu_ref, xi_ref, wu_ref, bu_ref, wi_ref, bi_ref,
                hu_ref, hi_ref):
    hu_ref[...] = (
        jnp.dot(xu_ref[...], wu_ref[...], preferred_element_type=jnp.float32)
        + bu_ref[...]
    )
    hi_ref[...] = (
        jnp.dot(xi_ref[...], wi_ref[...], preferred_element_type=jnp.float32)
        + bi_ref[...]
    )


def _embed(x_user, x_item, W_emb_user, b_emb_user, W_emb_item, b_emb_item):
    grid = (N // BLK,)
    row_spec = pl.BlockSpec((BLK, D), lambda i: (i, 0))
    full_spec = pl.BlockSpec((D, D), lambda i: (0, 0))
    bias_spec = pl.BlockSpec((1, D), lambda i: (0, 0))
    return pl.pallas_call(
        _embed_body,
        grid=grid,
        in_specs=[row_spec, row_spec, full_spec, bias_spec, full_spec,
                  bias_spec],
        out_specs=[row_spec, row_spec],
        out_shape=[
            jax.ShapeDtypeStruct((N, D), jnp.float32),
            jax.ShapeDtypeStruct((N, D), jnp.float32),
        ],
    )(x_user, x_item, W_emb_user, b_emb_user.reshape(1, D), W_emb_item,
      b_emb_item.reshape(1, D))


def _sc_agg_body(hu_hbm, hi_hbm, eidx_ui_hbm, eidx_iu_hbm,
                 agg_item_hbm, agg_user_hbm,
                 ib0, ib1, rows0, rows1, zbuf, acc,
                 is0, is1, gs0, gs1):
    core = lax.axis_index("c")
    sub = lax.axis_index("s")

    def _process(h_hbm, eidx_hbm, out_hbm):
        base_chunk = sub * NCHUNK

        def _ld_idx(g, buf, sem):
            pltpu.async_copy(eidx_hbm.at[base_chunk + g], buf, sem)

        def _ld_wait(buf, sem):
            pltpu.make_async_copy(eidx_hbm.at[base_chunk], buf, sem).wait()

        def _gather(buf, rbuf, sem):
            pltpu.async_copy(h_hbm.at[buf.at[0]], rbuf, sem)

        def _g_wait(rbuf, sem):
            pltpu.make_async_copy(h_hbm.at[ib0.at[0]], rbuf, sem).wait()

        def _scatter(rbuf, buf):
            pltpu.sync_copy(rbuf, acc.at[buf.at[1]], add=True)

        _ld_idx(0, ib0, is0)
        _ld_idx(1, ib1, is1)

        def _zero_buf(i, _):
            r = i // (D // 16)
            c = (i % (D // 16)) * 16
            zbuf[r, pl.ds(c, 16)] = jnp.zeros((16,), jnp.float32)
            return ()

        lax.fori_loop(0, ZR * (D // 16), _zero_buf, ())
        base_row = sub * ROWS_PER_TILE

        def _zero_acc(j, _):
            pltpu.sync_copy(zbuf, acc.at[pl.ds(base_row + j * ZR, ZR)])
            return ()

        lax.fori_loop(0, ROWS_PER_TILE // ZR, _zero_acc, ())
        plsc.subcore_barrier()

        _g_wait_0 = _g_wait
        _ld_wait(ib0, is0)
        _gather(ib0, rows0, gs0)

        def _pair(i, _):
            g = 2 * i
            _ld_wait(ib1, is1)
            _gather(ib1, rows1, gs1)
            _g_wait_0(rows0, gs0)
            _scatter(rows0, ib0)
            _ld_idx(jnp.minimum(g + 2, NCHUNK - 1), ib0, is0)
            _g_wait_0(rows1, gs1)
            _ld_wait(ib0, is0)
            _gather(ib0, rows0, gs0)
            _scatter(rows1, ib1)
            _ld_idx(jnp.minimum(g + 3, NCHUNK - 1), ib1, is1)
            return ()

        lax.fori_loop(0, NCHUNK // 2, _pair, ())
        _g_wait_0(rows0, gs0)
        _ld_wait(ib1, is1)
        plsc.subcore_barrier()

        pltpu.sync_copy(acc.at[pl.ds(base_row, ROWS_PER_TILE)],
                        out_hbm.at[pl.ds(base_row, ROWS_PER_TILE)])

    @pl.when(core == 0)
    def _():
        _process(hu_hbm, eidx_ui_hbm, agg_item_hbm)

    @pl.when(core == 1)
    def _():
        _process(hi_hbm, eidx_iu_hbm, agg_user_hbm)


def _sc_aggregate(h_user, h_item, src_ui, dst_ui, src_iu, dst_iu):
    mesh = plsc.VectorSubcoreMesh(core_axis_name="c", subcore_axis_name="s",
                                  num_cores=NC, num_subcores=NS)
    agg = pl.kernel(
        _sc_agg_body,
        out_type=[
            jax.ShapeDtypeStruct((N_PAD, D), jnp.float32),
            jax.ShapeDtypeStruct((N_PAD, D), jnp.float32),
        ],
        mesh=mesh,
        scratch_types=[
            pltpu.VMEM((2, K), jnp.int32),
            pltpu.VMEM((2, K), jnp.int32),
            pltpu.VMEM((K, D), jnp.float32),
            pltpu.VMEM((K, D), jnp.float32),
            pltpu.VMEM((ZR, D), jnp.float32),
            pltpu.VMEM_SHARED((N_PAD, D), jnp.float32),
            pltpu.SemaphoreType.DMA,
            pltpu.SemaphoreType.DMA,
            pltpu.SemaphoreType.DMA,
            pltpu.SemaphoreType.DMA,
        ],
    )

    def _pack(src, dst):
        return jnp.stack(
            [src.reshape(NS * NCHUNK, K), dst.reshape(NS * NCHUNK, K)],
            axis=1)

    return agg(h_user, h_item, _pack(src_ui, dst_ui), _pack(src_iu, dst_iu))


def _mlp_body(hi_ref, ai_ref, hu_ref, au_ref,
              w1ui_ref, b1ui_ref, w2ui_ref, b2ui_ref,
              w1iu_ref, b1iu_ref, w2iu_ref, b2iu_ref,
              oi_ref, ou_ref):
    zi = hi_ref[...] + ai_ref[...]
    ti = jnp.maximum(
        jnp.dot(zi, w1ui_ref[...], preferred_element_type=jnp.float32)
        + b1ui_ref[...], 0.0)
    oi_ref[...] = (
        jnp.dot(ti, w2ui_ref[...], preferred_element_type=jnp.float32)
        + b2ui_ref[...]
    )
    zu = hu_ref[...] + au_ref[...]
    tu = jnp.maximum(
        jnp.dot(zu, w1iu_ref[...], preferred_element_type=jnp.float32)
        + b1iu_ref[...], 0.0)
    ou_ref[...] = (
        jnp.dot(tu, w2iu_ref[...], preferred_element_type=jnp.float32)
        + b2iu_ref[...]
    )


def _mlp(h_item, agg_item, h_user, agg_user,
         W1_ui, b1_ui, W2_ui, b2_ui, W1_iu, b1_iu, W2_iu, b2_iu):
    grid = (N // BLK,)
    row_spec = pl.BlockSpec((BLK, D), lambda i: (i, 0))
    full_spec = pl.BlockSpec((D, D), lambda i: (0, 0))
    bias_spec = pl.BlockSpec((1, D), lambda i: (0, 0))
    return pl.pallas_call(
        _mlp_body,
        grid=grid,
        in_specs=[row_spec, row_spec, row_spec, row_spec,
                  full_spec, bias_spec, full_spec, bias_spec,
                  full_spec, bias_spec, full_spec, bias_spec],
        out_specs=[row_spec, row_spec],
        out_shape=[
            jax.ShapeDtypeStruct((N, D), jnp.float32),
            jax.ShapeDtypeStruct((N, D), jnp.float32),
        ],
    )(h_item, agg_item, h_user, agg_user,
      W1_ui, b1_ui.reshape(1, D), W2_ui, b2_ui.reshape(1, D),
      W1_iu, b1_iu.reshape(1, D), W2_iu, b2_iu.reshape(1, D))


def kernel(x_user, x_item, edge_index_user_item, edge_index_item_user,
           W_emb_user, b_emb_user, W_emb_item, b_emb_item,
           W1_ui, b1_ui, W2_ui, b2_ui, W1_iu, b1_iu, W2_iu, b2_iu):
    src_ui = edge_index_user_item[0]
    dst_ui = edge_index_user_item[1]
    src_iu = edge_index_item_user[0]
    dst_iu = edge_index_item_user[1]

    h_user, h_item = _embed(x_user, x_item, W_emb_user, b_emb_user,
                            W_emb_item, b_emb_item)
    agg_item, agg_user = _sc_aggregate(h_user, h_item, src_ui, dst_ui,
                                       src_iu, dst_iu)
    out_item, out_user = _mlp(h_item, agg_item, h_user, agg_user,
                              W1_ui, b1_ui, W2_ui, b2_ui,
                              W1_iu, b1_iu, W2_iu, b2_iu)
    return (out_user, out_item)


# speedup vs baseline: 12.4747x; 1.2679x over previous
import functools

import jax
import jax.numpy as jnp
from jax import lax
from jax.experimental import pallas as pl
from jax.experimental.pallas import tpu as pltpu
from jax.experimental.pallas import tpu_sc as plsc

N = 10000
D = 128
E = 320000

NC = 2
NS = 16
ET = E // NS
K = 125
NCHUNK = ET // K
B = 8
NBLK = NCHUNK // B
N_PAD = 10240
ROWS_PER_TILE = N_PAD // NS
ZR = 32


BLK = 2000


def _embed_body(xu_ref, xi_ref, wu_ref, bu_ref, wi_ref, bi_ref,
                hu_ref, hi_ref):
    hu_ref[...] = (
        jnp.dot(xu_ref[...], wu_ref[...], preferred_element_type=jnp.float32)
        + bu_ref[...]
    )
    hi_ref[...] = (
        jnp.dot(xi_ref[...], wi_ref[...], preferred_element_type=jnp.float32)
        + bi_ref[...]
    )


def _embed(x_user, x_item, W_emb_user, b_emb_user, W_emb_item, b_emb_item):
    grid = (N // BLK,)
    row_spec = pl.BlockSpec((BLK, D), lambda i: (i, 0))
    full_spec = pl.BlockSpec((D, D), lambda i: (0, 0))
    bias_spec = pl.BlockSpec((1, D), lambda i: (0, 0))
    return pl.pallas_call(
        _embed_body,
        grid=grid,
        in_specs=[row_spec, row_spec, full_spec, bias_spec, full_spec,
                  bias_spec],
        out_specs=[row_spec, row_spec],
        out_shape=[
            jax.ShapeDtypeStruct((N, D), jnp.float32),
            jax.ShapeDtypeStruct((N, D), jnp.float32),
        ],
    )(x_user, x_item, W_emb_user, b_emb_user.reshape(1, D), W_emb_item,
      b_emb_item.reshape(1, D))


def _sc_agg_body(hu_hbm, hi_hbm, eidx_ui_hbm, eidx_iu_hbm,
                 agg_item_hbm, agg_user_hbm,
                 sblk0, sblk1, dblk0, dblk1, rows0, rows1, zbuf, acc,
                 is0, is1, id0, id1, gs0, gs1):
    core = lax.axis_index("c")
    sub = lax.axis_index("s")
    rows = (rows0, rows1)
    gs = (gs0, gs1)

    def _process(h_hbm, eidx_hbm, out_hbm):
        base_chunk = sub * NCHUNK

        def _ld_blk(bi, sbuf, dbuf, ssem, dsem):
            gg = pl.multiple_of(base_chunk + bi * B, 8)
            pltpu.async_copy(eidx_hbm.at[0, pl.ds(gg, B)], sbuf, ssem)
            pltpu.async_copy(eidx_hbm.at[1, pl.ds(gg, B)], dbuf, dsem)

        def _blk_wait(buf, sem):
            pltpu.make_async_copy(eidx_hbm.at[0, pl.ds(base_chunk, B)],
                                  buf, sem).wait()

        def _gather(idx_ref, rbuf, sem):
            pltpu.async_copy(h_hbm.at[idx_ref], rbuf, sem)

        def _g_wait(rbuf, sem):
            pltpu.make_async_copy(h_hbm.at[sblk0.at[0]], rbuf, sem).wait()

        def _scatter(rbuf, idx_ref):
            pltpu.sync_copy(rbuf, acc.at[idx_ref], add=True)

        _ld_blk(0, sblk0, dblk0, is0, id0)
        _ld_blk(1, sblk1, dblk1, is1, id1)

        def _zero_buf(i, _):
            r = i // (D // 16)
            c = (i % (D // 16)) * 16
            zbuf[r, pl.ds(c, 16)] = jnp.zeros((16,), jnp.float32)
            return ()

        lax.fori_loop(0, ZR * (D // 16), _zero_buf, ())
        base_row = sub * ROWS_PER_TILE

        def _zero_acc(j, _):
            pltpu.sync_copy(zbuf, acc.at[pl.ds(base_row + j * ZR, ZR)])
            return ()

        lax.fori_loop(0, ROWS_PER_TILE // ZR, _zero_acc, ())
        plsc.subcore_barrier()

        _blk_wait(sblk0, is0)
        _gather(sblk0.at[0], rows0, gs0)

        def _iter(i, _):
            nblk2 = jnp.minimum(2 * i + 2, NBLK - 1)
            nblk3 = jnp.minimum(2 * i + 3, NBLK - 1)
            for j in range(2 * B):
                p = j % 2
                q = (j + 1) % 2
                in0 = j < B
                sblk_n = sblk0 if j + 1 < B else sblk1
                if j + 1 == B:
                    _blk_wait(sblk1, is1)
                if j + 1 == 2 * B:
                    _blk_wait(sblk0, is0)
                    _gather(sblk0.at[0], rows[0], gs[0])
                else:
                    _gather(sblk_n.at[(j + 1) % B], rows[q], gs[q])
                if j == 0:
                    _blk_wait(dblk0, id0)
                if j == B:
                    _blk_wait(dblk1, id1)
                _g_wait(rows[p], gs[p])
                _scatter(rows[p], (dblk0 if in0 else dblk1).at[j % B])
                if j == B - 1:
                    _ld_blk(nblk2, sblk0, dblk0, is0, id0)
                if j == 2 * B - 1:
                    _ld_blk(nblk3, sblk1, dblk1, is1, id1)
            return ()

        lax.fori_loop(0, NBLK // 2, _iter, ())
        _g_wait(rows0, gs0)
        _blk_wait(sblk1, is1)
        _blk_wait(dblk1, id1)
        _blk_wait(dblk0, id0)
        plsc.subcore_barrier()

        pltpu.sync_copy(acc.at[pl.ds(base_row, ROWS_PER_TILE)],
                        out_hbm.at[pl.ds(base_row, ROWS_PER_TILE)])

    @pl.when(core == 0)
    def _():
        _process(hu_hbm, eidx_ui_hbm, agg_item_hbm)

    @pl.when(core == 1)
    def _():
        _process(hi_hbm, eidx_iu_hbm, agg_user_hbm)


def _sc_aggregate(h_user, h_item, edge_ui, edge_iu):
    mesh = plsc.VectorSubcoreMesh(core_axis_name="c", subcore_axis_name="s",
                                  num_cores=NC, num_subcores=NS)
    agg = pl.kernel(
        _sc_agg_body,
        out_type=[
            jax.ShapeDtypeStruct((N_PAD, D), jnp.float32),
            jax.ShapeDtypeStruct((N_PAD, D), jnp.float32),
        ],
        mesh=mesh,
        scratch_types=[
            pltpu.VMEM((B, K), jnp.int32),
            pltpu.VMEM((B, K), jnp.int32),
            pltpu.VMEM((B, K), jnp.int32),
            pltpu.VMEM((B, K), jnp.int32),
            pltpu.VMEM((K, D), jnp.float32),
            pltpu.VMEM((K, D), jnp.float32),
            pltpu.VMEM((ZR, D), jnp.float32),
            pltpu.VMEM_SHARED((N_PAD, D), jnp.float32),
            pltpu.SemaphoreType.DMA,
            pltpu.SemaphoreType.DMA,
            pltpu.SemaphoreType.DMA,
            pltpu.SemaphoreType.DMA,
            pltpu.SemaphoreType.DMA,
            pltpu.SemaphoreType.DMA,
        ],
    )

    def _view(eidx):
        return eidx.reshape(2, NS * NCHUNK, K)

    return agg(h_user, h_item, _view(edge_ui), _view(edge_iu))


def _mlp_body(hi_ref, ai_ref, hu_ref, au_ref,
              w1ui_ref, b1ui_ref, w2ui_ref, b2ui_ref,
              w1iu_ref, b1iu_ref, w2iu_ref, b2iu_ref,
              oi_ref, ou_ref):
    zi = hi_ref[...] + ai_ref[...]
    ti = jnp.maximum(
        jnp.dot(zi, w1ui_ref[...], preferred_element_type=jnp.float32)
        + b1ui_ref[...], 0.0)
    oi_ref[...] = (
        jnp.dot(ti, w2ui_ref[...], preferred_element_type=jnp.float32)
        + b2ui_ref[...]
    )
    zu = hu_ref[...] + au_ref[...]
    tu = jnp.maximum(
        jnp.dot(zu, w1iu_ref[...], preferred_element_type=jnp.float32)
        + b1iu_ref[...], 0.0)
    ou_ref[...] = (
        jnp.dot(tu, w2iu_ref[...], preferred_element_type=jnp.float32)
        + b2iu_ref[...]
    )


def _mlp(h_item, agg_item, h_user, agg_user,
         W1_ui, b1_ui, W2_ui, b2_ui, W1_iu, b1_iu, W2_iu, b2_iu):
    grid = (N // BLK,)
    row_spec = pl.BlockSpec((BLK, D), lambda i: (i, 0))
    full_spec = pl.BlockSpec((D, D), lambda i: (0, 0))
    bias_spec = pl.BlockSpec((1, D), lambda i: (0, 0))
    return pl.pallas_call(
        _mlp_body,
        grid=grid,
        in_specs=[row_spec, row_spec, row_spec, row_spec,
                  full_spec, bias_spec, full_spec, bias_spec,
                  full_spec, bias_spec, full_spec, bias_spec],
        out_specs=[row_spec, row_spec],
        out_shape=[
            jax.ShapeDtypeStruct((N, D), jnp.float32),
            jax.ShapeDtypeStruct((N, D), jnp.float32),
        ],
    )(h_item, agg_item, h_user, agg_user,
      W1_ui, b1_ui.reshape(1, D), W2_ui, b2_ui.reshape(1, D),
      W1_iu, b1_iu.reshape(1, D), W2_iu, b2_iu.reshape(1, D))


def kernel(x_user, x_item, edge_index_user_item, edge_index_item_user,
           W_emb_user, b_emb_user, W_emb_item, b_emb_item,
           W1_ui, b1_ui, W2_ui, b2_ui, W1_iu, b1_iu, W2_iu, b2_iu):
    h_user, h_item = _embed(x_user, x_item, W_emb_user, b_emb_user,
                            W_emb_item, b_emb_item)
    agg_item, agg_user = _sc_aggregate(h_user, h_item,
                                       edge_index_user_item,
                                       edge_index_item_user)
    out_item, out_user = _mlp(h_item, agg_item, h_user, agg_user,
                              W1_ui, b1_ui, W2_ui, b2_ui,
                              W1_iu, b1_iu, W2_iu, b2_iu)
    return (out_user, out_item)
